# baseline (device time: 9882 ns/iter reference)
import jax
import jax.numpy as jnp
from jax import lax
from jax.experimental import pallas as pl
from jax.experimental.pallas import tpu as pltpu

N_DEV = 8


def kernel(x):
    m_rows, n_cols = x.shape

    def body(x_ref, out_ref, e_ref, gather_ref, send_sems, recv_sems):
        my_pos = lax.axis_index("i")

        barrier_sem = pltpu.get_barrier_semaphore()
        for k in range(1, N_DEV):
            pl.semaphore_signal(
                barrier_sem, inc=1,
                device_id=(my_pos ^ k,), device_id_type=pl.DeviceIdType.MESH,
            )

        xv = x_ref[:, :]
        m = jnp.max(xv, axis=1)
        e = jnp.exp(xv - m[:, None])
        s = jnp.sum(e, axis=1)
        gather_ref[0] = jnp.stack([m, s], axis=0)

        pl.semaphore_wait(barrier_sem, N_DEV - 1)

        sends = []
        for k in range(1, N_DEV):
            rdma = pltpu.make_async_remote_copy(
                src_ref=gather_ref.at[0],
                dst_ref=gather_ref.at[k],
                send_sem=send_sems.at[k],
                recv_sem=recv_sems.at[k],
                device_id=(my_pos ^ k,),
                device_id_type=pl.DeviceIdType.MESH,
            )
            rdma.start()
            sends.append(rdma)

        e_ref[:, :] = e.astype(jnp.bfloat16)

        for rdma in sends:
            rdma.wait_recv()

        all_m = gather_ref[:, 0, :]
        all_s = gather_ref[:, 1, :]
        gmax = jnp.max(all_m, axis=0)
        gsum = jnp.sum(all_s * jnp.exp(all_m - gmax[None, :]), axis=0)
        scale = jnp.exp(m - gmax) / gsum
        out_ref[:, :] = e_ref[:, :] * scale.astype(jnp.bfloat16)[:, None]

        for rdma in sends:
            rdma.wait_send()

    return pl.pallas_call(
        body,
        out_shape=jax.ShapeDtypeStruct((m_rows, n_cols), jnp.bfloat16),
        in_specs=[pl.BlockSpec(memory_space=pltpu.VMEM)],
        out_specs=pl.BlockSpec(memory_space=pltpu.VMEM),
        scratch_shapes=[
            pltpu.VMEM((m_rows, n_cols), jnp.bfloat16),
            pltpu.VMEM((N_DEV, 2, m_rows), jnp.float32),
            pltpu.SemaphoreType.DMA((N_DEV,)),
            pltpu.SemaphoreType.DMA((N_DEV,)),
        ],
        compiler_params=pltpu.CompilerParams(collective_id=0),
    )(x)
